# baseline (speedup 1.0000x reference)
import functools

import jax
import jax.numpy as jnp
from jax import lax
from jax.experimental import pallas as pl
from jax.experimental.pallas import tpu as pltpu
from jax.experimental.pallas import tpu_sc as plsc

N = 2048
IN_SZ = 128
OUT_SZ = 64
OUT_BASE = N - OUT_SZ
L = 16
NS = 16
UN = 2


def _tanh16(x):
    ex = jnp.exp(-2.0 * jnp.abs(x))
    return jnp.sign(x) * (1.0 - ex) / (1.0 + ex)


def _brain_sc(edge_index, weights, input_data, biases, zeros):
    e = weights.shape[0]
    batch = input_data.shape[0]
    e_per = e // NS
    chunks = e_per // L
    act_sz = batch * N
    out_sz = batch * OUT_SZ
    red_per = act_sz // NS
    outred_per = out_sz // NS
    mesh = plsc.VectorSubcoreMesh(
        core_axis_name="c", subcore_axis_name="s", num_cores=1, num_subcores=NS
    )

    @functools.partial(
        pl.kernel,
        out_type=jax.ShapeDtypeStruct((batch, OUT_SZ), jnp.float32),
        mesh=mesh,
        compiler_params=pltpu.CompilerParams(needs_layout_passes=False),
        scratch_types=[
            pltpu.VMEM((e_per,), jnp.int32),
            pltpu.VMEM((e_per,), jnp.int32),
            pltpu.VMEM((e_per,), jnp.float32),
            pltpu.VMEM((batch, IN_SZ), jnp.float32),
            pltpu.VMEM((act_sz,), jnp.float32),
            pltpu.VMEM((act_sz,), jnp.float32),
            pltpu.VMEM((NS, red_per), jnp.float32),
            pltpu.VMEM((red_per,), jnp.float32),
            pltpu.VMEM((out_sz,), jnp.float32),
            pltpu.VMEM((NS * out_sz,), jnp.float32),
            pltpu.VMEM((OUT_SZ,), jnp.float32),
            pltpu.VMEM_SHARED((NS, act_sz), jnp.float32),
            pltpu.VMEM_SHARED((act_sz,), jnp.float32),
            pltpu.VMEM_SHARED((NS * out_sz,), jnp.float32),
            pltpu.SemaphoreType.DMA,
        ],
    )
    def run(edge_hbm, w_hbm, in_hbm, b_hbm, z_hbm, out_hbm,
            src_v, dst_v, w_v, in_v, part_v, act1_v, red_v, slice_v,
            outp_v, outred_v, fin_v, parts_sh, act1_sh, outparts_sh, sem):
        tid = lax.axis_index("s")
        ebase = tid * e_per
        rbase = tid * red_per

        copies = [
            pltpu.async_copy(edge_hbm.at[0, pl.ds(ebase, e_per)], src_v, sem),
            pltpu.async_copy(edge_hbm.at[1, pl.ds(ebase, e_per)], dst_v, sem),
            pltpu.async_copy(w_hbm.at[pl.ds(ebase, e_per)], w_v, sem),
            pltpu.async_copy(in_hbm, in_v, sem),
            pltpu.async_copy(z_hbm.at[pl.ds(0, act_sz)], part_v, sem),
            pltpu.async_copy(z_hbm.at[pl.ds(0, out_sz)], outp_v, sem),
            pltpu.async_copy(b_hbm.at[pl.ds(rbase % N, red_per)], slice_v, sem),
        ]
        with jax.named_scope("stage_wait"):
            for c in copies:
                c.wait()

        def one_chunk1(i):
            s = src_v[pl.ds(i * L, L)]
            d = dst_v[pl.ds(i * L, L)]
            w = w_v[pl.ds(i * L, L)]
            m = s < IN_SZ
            s_c = jnp.where(m, s, 0)
            for bb in range(batch):
                row = jnp.full((L,), bb, jnp.int32)
                val = plsc.load_gather(in_v, [row, s_c]) * w
                plsc.addupdate_scatter(part_v, [d + (bb * N)], val, mask=m)

        def step1(i, _):
            for u in range(UN):
                one_chunk1(i * UN + u)
            return 0

        with jax.named_scope("step1"):
            lax.fori_loop(0, chunks // UN, step1, 0)

        with jax.named_scope("publish1"):
            pltpu.sync_copy(part_v, parts_sh.at[tid])
            plsc.subcore_barrier()
            pltpu.sync_copy(parts_sh.at[:, pl.ds(rbase, red_per)], red_v)

        def reduce1(i, _):
            acc = red_v[0, pl.ds(i * L, L)]
            for p in range(1, NS):
                acc = acc + red_v[p, pl.ds(i * L, L)]
            slice_v[pl.ds(i * L, L)] = _tanh16(acc + slice_v[pl.ds(i * L, L)])
            return 0

        with jax.named_scope("reduce1"):
            lax.fori_loop(0, red_per // L, reduce1, 0)

        with jax.named_scope("act1x"):
            pltpu.sync_copy(slice_v, act1_sh.at[pl.ds(rbase, red_per)])
            plsc.subcore_barrier()
            pltpu.sync_copy(act1_sh, act1_v)

        def one_chunk2(i):
            s = src_v[pl.ds(i * L, L)]
            d = dst_v[pl.ds(i * L, L)]
            w = w_v[pl.ds(i * L, L)]
            m = d >= OUT_BASE
            j = jnp.where(m, d - OUT_BASE, 0)
            for bb in range(batch):
                val = plsc.load_gather(act1_v, [s + (bb * N)]) * w
                plsc.addupdate_scatter(outp_v, [j + (bb * OUT_SZ)], val, mask=m)

        def step2(i, _):
            for u in range(UN):
                one_chunk2(i * UN + u)
            return 0

        with jax.named_scope("step2"):
            lax.fori_loop(0, chunks // UN, step2, 0)

        pltpu.sync_copy(outp_v, outparts_sh.at[pl.ds(tid * out_sz, out_sz)])
        plsc.subcore_barrier()

        @pl.when(tid < batch)
        def _():
            pltpu.sync_copy(outparts_sh, outred_v)
            pltpu.sync_copy(b_hbm.at[pl.ds(OUT_BASE, OUT_SZ)],
                            fin_v)
            obase = tid * OUT_SZ
            for i in range(OUT_SZ // L):
                acc = fin_v[pl.ds(i * L, L)]
                for p in range(NS):
                    acc = acc + outred_v[pl.ds(p * out_sz + obase + i * L, L)]
                fin_v[pl.ds(i * L, L)] = _tanh16(acc)
            pltpu.sync_copy(fin_v, out_hbm.at[tid])

    return run(edge_index, weights, input_data, biases, zeros)


def kernel(input_data, edge_index, connection_weights, biases):
    b = input_data.shape[0]
    zeros = jnp.zeros((b * N,), jnp.float32)
    return _brain_sc(edge_index, connection_weights, input_data, biases, zeros)


# speedup vs baseline: 1.1083x; 1.1083x over previous
import functools

import jax
import jax.numpy as jnp
from jax import lax
from jax.experimental import pallas as pl
from jax.experimental.pallas import tpu as pltpu
from jax.experimental.pallas import tpu_sc as plsc

N = 2048
IN_SZ = 128
OUT_SZ = 64
OUT_BASE = N - OUT_SZ
L = 16
NS = 16


def _tanh16(x):
    ex = jnp.exp(-2.0 * jnp.abs(x))
    return jnp.sign(x) * (1.0 - ex) / (1.0 + ex)


def _brain_sc(edge_index, weights, input_data, biases, zeros):
    e = weights.shape[0]
    batch = input_data.shape[0]
    e_per = e // NS
    chunks = e_per // L
    act_sz = batch * N
    out_sz = batch * OUT_SZ
    red_per = act_sz // NS
    outred_per = out_sz // NS
    mesh = plsc.VectorSubcoreMesh(
        core_axis_name="c", subcore_axis_name="s", num_cores=1, num_subcores=NS
    )

    @functools.partial(
        pl.kernel,
        out_type=jax.ShapeDtypeStruct((batch, OUT_SZ), jnp.float32),
        mesh=mesh,
        compiler_params=pltpu.CompilerParams(needs_layout_passes=False),
        scratch_types=[
            pltpu.VMEM((e_per,), jnp.int32),
            pltpu.VMEM((e_per,), jnp.int32),
            pltpu.VMEM((e_per,), jnp.float32),
            pltpu.VMEM((batch, IN_SZ), jnp.float32),
            pltpu.VMEM((act_sz,), jnp.float32),
            pltpu.VMEM((act_sz,), jnp.float32),
            pltpu.VMEM((NS, red_per), jnp.float32),
            pltpu.VMEM((red_per,), jnp.float32),
            pltpu.VMEM((out_sz,), jnp.float32),
            pltpu.VMEM((NS * out_sz,), jnp.float32),
            pltpu.VMEM((OUT_SZ,), jnp.float32),
            pltpu.VMEM((e_per + L,), jnp.int32),
            pltpu.VMEM((e_per + L,), jnp.int32),
            pltpu.VMEM((e_per + L,), jnp.float32),
            pltpu.VMEM((e_per + L,), jnp.int32),
            pltpu.VMEM((e_per + L,), jnp.int32),
            pltpu.VMEM((e_per + L,), jnp.float32),
            pltpu.VMEM_SHARED((NS, act_sz), jnp.float32),
            pltpu.VMEM_SHARED((act_sz,), jnp.float32),
            pltpu.VMEM_SHARED((NS * out_sz,), jnp.float32),
            pltpu.SemaphoreType.DMA,
        ],
    )
    def run(edge_hbm, w_hbm, in_hbm, b_hbm, z_hbm, out_hbm,
            src_v, dst_v, w_v, in_v, part_v, act1_v, red_v, slice_v,
            outp_v, outred_v, fin_v, cs1, cd1, cw1, cs2, cd2, cw2,
            parts_sh, act1_sh, outparts_sh, sem):
        tid = lax.axis_index("s")
        ebase = tid * e_per
        rbase = tid * red_per

        copies = [
            pltpu.async_copy(edge_hbm.at[0, pl.ds(ebase, e_per)], src_v, sem),
            pltpu.async_copy(edge_hbm.at[1, pl.ds(ebase, e_per)], dst_v, sem),
            pltpu.async_copy(w_hbm.at[pl.ds(ebase, e_per)], w_v, sem),
            pltpu.async_copy(in_hbm, in_v, sem),
            pltpu.async_copy(z_hbm.at[pl.ds(0, act_sz)], part_v, sem),
            pltpu.async_copy(z_hbm.at[pl.ds(0, out_sz)], outp_v, sem),
            pltpu.async_copy(b_hbm.at[pl.ds(rbase % N, red_per)], slice_v, sem),
        ]
        with jax.named_scope("stage_wait"):
            for c in copies:
                c.wait()

        def classify(i, carry):
            c1, c2 = carry
            s = src_v[pl.ds(i * L, L)]
            d = dst_v[pl.ds(i * L, L)]
            w = w_v[pl.ds(i * L, L)]
            m1 = s < IN_SZ
            plsc.store_compressed(cs1.at[pl.ds(c1, L)], s, mask=m1)
            plsc.store_compressed(cd1.at[pl.ds(c1, L)], d, mask=m1)
            plsc.store_compressed(cw1.at[pl.ds(c1, L)], w, mask=m1)
            m2 = d >= OUT_BASE
            plsc.store_compressed(cs2.at[pl.ds(c2, L)], s, mask=m2)
            plsc.store_compressed(cd2.at[pl.ds(c2, L)], d, mask=m2)
            plsc.store_compressed(cw2.at[pl.ds(c2, L)], w, mask=m2)
            n1 = jnp.max(plsc.all_reduce_population_count(m1))
            n2 = jnp.max(plsc.all_reduce_population_count(m2))
            return (c1 + n1, c2 + n2)

        with jax.named_scope("classify"):
            c1, c2 = lax.fori_loop(
                0, chunks, classify, (jnp.int32(0), jnp.int32(0))
            )

        lanes = jnp.arange(L, dtype=jnp.int32)

        def step1(i, _):
            m = lanes < (c1 - i * L)
            s = jnp.where(m, cs1[pl.ds(i * L, L)], 0)
            d = jnp.where(m, cd1[pl.ds(i * L, L)], 0)
            w = cw1[pl.ds(i * L, L)]
            for bb in range(batch):
                row = jnp.full((L,), bb, jnp.int32)
                val = plsc.load_gather(in_v, [row, s]) * w
                plsc.addupdate_scatter(part_v, [d + (bb * N)], val, mask=m)
            return 0

        with jax.named_scope("step1"):
            lax.fori_loop(0, (c1 + L - 1) // L, step1, 0)

        with jax.named_scope("publish1"):
            pltpu.sync_copy(part_v, parts_sh.at[tid])
            plsc.subcore_barrier()
            pltpu.sync_copy(parts_sh.at[:, pl.ds(rbase, red_per)], red_v)

        def reduce1(i, _):
            acc = red_v[0, pl.ds(i * L, L)]
            for p in range(1, NS):
                acc = acc + red_v[p, pl.ds(i * L, L)]
            slice_v[pl.ds(i * L, L)] = _tanh16(acc + slice_v[pl.ds(i * L, L)])
            return 0

        with jax.named_scope("reduce1"):
            lax.fori_loop(0, red_per // L, reduce1, 0)

        with jax.named_scope("act1x"):
            pltpu.sync_copy(slice_v, act1_sh.at[pl.ds(rbase, red_per)])
            plsc.subcore_barrier()
            pltpu.sync_copy(act1_sh, act1_v)

        def step2(i, _):
            m = lanes < (c2 - i * L)
            s = jnp.where(m, cs2[pl.ds(i * L, L)], 0)
            j = jnp.where(m, cd2[pl.ds(i * L, L)] - OUT_BASE, 0)
            w = cw2[pl.ds(i * L, L)]
            for bb in range(batch):
                val = plsc.load_gather(act1_v, [s + (bb * N)]) * w
                plsc.addupdate_scatter(outp_v, [j + (bb * OUT_SZ)], val, mask=m)
            return 0

        with jax.named_scope("step2"):
            lax.fori_loop(0, (c2 + L - 1) // L, step2, 0)

        pltpu.sync_copy(outp_v, outparts_sh.at[pl.ds(tid * out_sz, out_sz)])
        plsc.subcore_barrier()

        @pl.when(tid < batch)
        def _():
            pltpu.sync_copy(outparts_sh, outred_v)
            pltpu.sync_copy(b_hbm.at[pl.ds(OUT_BASE, OUT_SZ)],
                            fin_v)
            obase = tid * OUT_SZ
            for i in range(OUT_SZ // L):
                acc = fin_v[pl.ds(i * L, L)]
                for p in range(NS):
                    acc = acc + outred_v[pl.ds(p * out_sz + obase + i * L, L)]
                fin_v[pl.ds(i * L, L)] = _tanh16(acc)
            pltpu.sync_copy(fin_v, out_hbm.at[tid])

    return run(edge_index, weights, input_data, biases, zeros)


def kernel(input_data, edge_index, connection_weights, biases):
    b = input_data.shape[0]
    zeros = jnp.zeros((b * N,), jnp.float32)
    return _brain_sc(edge_index, connection_weights, input_data, biases, zeros)
